# Initial kernel scaffold; baseline (speedup 1.0000x reference)
import functools

import jax
import jax.numpy as jnp
from jax import lax
from jax.experimental import pallas as pl
from jax.experimental.pallas import tpu as pltpu
from jax.experimental.pallas import tpu_sc as plsc

NC, NS, LANES = 2, 16, 16
NW = NC * NS
EB = 128
NB = 80
E_PAD = NW * NB * EB
N_NODES = 10000
RPT = N_NODES // NS
FC = 128


def _sc_agg(h, src3, dst3, ew3, zeros):
    mesh = plsc.VectorSubcoreMesh(
        core_axis_name="c", subcore_axis_name="s", num_cores=NC, num_subcores=NS)

    @functools.partial(
        pl.kernel,
        out_type=jax.ShapeDtypeStruct((NC, N_NODES, FC), jnp.float32),
        mesh=mesh,
        scratch_types=[
            pltpu.VMEM((NB, EB), jnp.int32),
            pltpu.VMEM((NB, EB), jnp.int32),
            pltpu.VMEM((NB, EB), jnp.float32),
            pltpu.VMEM((EB, FC), jnp.float32),
            pltpu.VMEM_SHARED((N_NODES, FC), jnp.float32),
            pltpu.SemaphoreType.DMA,
        ],
    )
    def k(h_hbm, src_hbm, dst_hbm, ew_hbm, z_hbm, out_hbm,
          src_v, dst_v, ew_v, rows_v, acc_sh, sem):
        cid = lax.axis_index("c")
        sid = lax.axis_index("s")
        wid = cid * NS + sid
        pltpu.sync_copy(src_hbm.at[wid], src_v)
        pltpu.sync_copy(dst_hbm.at[wid], dst_v)
        pltpu.sync_copy(ew_hbm.at[wid], ew_v)
        r0 = sid * RPT
        pltpu.sync_copy(z_hbm.at[pl.ds(r0, RPT)], acc_sh.at[pl.ds(r0, RPT)])
        plsc.subcore_barrier()

        def batch_body(b, carry):
            pltpu.async_copy(h_hbm.at[src_v.at[b]], rows_v, sem).wait()

            def e16_body(e16, c2):
                wv = ew_v[b, pl.ds(e16 * LANES, LANES)]
                for i in range(LANES):
                    e = e16 * LANES + i
                    w16 = jnp.broadcast_to(wv[i], (LANES,))
                    for j in range(FC // LANES):
                        sl = pl.ds(j * LANES, LANES)
                        rows_v[e, sl] = rows_v[e, sl] * w16
                return c2

            lax.fori_loop(0, EB // LANES, e16_body, 0)
            pltpu.sync_copy(rows_v, acc_sh.at[dst_v.at[b]], add=True)
            return carry

        lax.fori_loop(0, NB, batch_body, 0)
        plsc.subcore_barrier()
        pltpu.sync_copy(acc_sh.at[pl.ds(r0, RPT)],
                        out_hbm.at[cid, pl.ds(r0, RPT)])

    return k(h, src3, dst3, ew3, zeros)


def _dense2(parts, h, wa_t, wb_t, b2d, relu, bn=500):
    n, fin = h.shape
    fout = wa_t.shape[1]

    def body(p_ref, h_ref, wa_ref, wb_ref, b_ref, o_ref):
        agg = p_ref[0] + p_ref[1]
        z = jnp.dot(agg, wa_ref[...], preferred_element_type=jnp.float32,
                    precision=lax.Precision.HIGHEST)
        z = z + jnp.dot(h_ref[...], wb_ref[...], preferred_element_type=jnp.float32,
                        precision=lax.Precision.HIGHEST)
        z = z + b_ref[...]
        o_ref[...] = jnp.maximum(z, 0.0) if relu else z

    return pl.pallas_call(
        body,
        grid=(n // bn,),
        in_specs=[
            pl.BlockSpec((2, bn, fin), lambda i: (0, i, 0)),
            pl.BlockSpec((bn, fin), lambda i: (i, 0)),
            pl.BlockSpec((fin, fout), lambda i: (0, 0)),
            pl.BlockSpec((fin, fout), lambda i: (0, 0)),
            pl.BlockSpec((1, fout), lambda i: (0, 0)),
        ],
        out_specs=pl.BlockSpec((bn, fout), lambda i: (i, 0)),
        out_shape=jax.ShapeDtypeStruct((n, fout), jnp.float32),
    )(parts, h, wa_t, wb_t, b2d)


def _matmul(h, w_t, bn=500):
    n, fin = h.shape
    fout = w_t.shape[1]

    def body(h_ref, w_ref, o_ref):
        o_ref[...] = jnp.dot(h_ref[...], w_ref[...],
                             preferred_element_type=jnp.float32,
                             precision=lax.Precision.HIGHEST)

    return pl.pallas_call(
        body,
        grid=(n // bn,),
        in_specs=[
            pl.BlockSpec((bn, fin), lambda i: (i, 0)),
            pl.BlockSpec((fin, fout), lambda i: (0, 0)),
        ],
        out_specs=pl.BlockSpec((bn, fout), lambda i: (i, 0)),
        out_shape=jax.ShapeDtypeStruct((n, fout), jnp.float32),
    )(h, w_t)


def _final(parts, h, w_t, b2d, bn=500):
    n, fin = h.shape
    fout = w_t.shape[1]

    def body(p_ref, h_ref, w_ref, b_ref, o_ref):
        z = jnp.dot(h_ref[...], w_ref[...], preferred_element_type=jnp.float32,
                    precision=lax.Precision.HIGHEST)
        o_ref[...] = p_ref[0] + p_ref[1] + z + b_ref[...]

    return pl.pallas_call(
        body,
        grid=(n // bn,),
        in_specs=[
            pl.BlockSpec((2, bn, fout), lambda i: (0, i, 0)),
            pl.BlockSpec((bn, fin), lambda i: (i, 0)),
            pl.BlockSpec((fin, fout), lambda i: (0, 0)),
            pl.BlockSpec((1, fout), lambda i: (0, 0)),
        ],
        out_specs=pl.BlockSpec((bn, fout), lambda i: (i, 0)),
        out_shape=jax.ShapeDtypeStruct((n, fout), jnp.float32),
    )(parts, h, w_t, b2d)


def kernel(x, edge_index, edge_weight, W1_rel, b1, W1_root,
           W2_rel, b2, W2_root, W3_rel, b3, W3_root):
    src = edge_index[0]
    dst = edge_index[1]
    pad = E_PAD - src.shape[0]
    src3 = jnp.concatenate([src, jnp.zeros((pad,), src.dtype)]).reshape(NW, NB, EB)
    dst3 = jnp.concatenate([dst, jnp.zeros((pad,), dst.dtype)]).reshape(NW, NB, EB)
    ew3 = jnp.concatenate([edge_weight, jnp.zeros((pad,), edge_weight.dtype)]
                          ).reshape(NW, NB, EB)
    zeros = jnp.zeros((N_NODES, FC), jnp.float32)

    p1 = _sc_agg(x, src3, dst3, ew3, zeros)
    h1 = _dense2(p1, x, W1_rel.T, W1_root.T, b1.reshape(1, -1), relu=True)

    p2a = _sc_agg(h1[:, :FC], src3, dst3, ew3, zeros)
    p2b = _sc_agg(h1[:, FC:], src3, dst3, ew3, zeros)
    parts2 = jnp.concatenate([p2a, p2b], axis=2)
    h2 = _dense2(parts2, h1, W2_rel.T, W2_root.T, b2.reshape(1, -1), relu=True)

    y3 = _matmul(h2, W3_rel.T)
    p3 = _sc_agg(y3, src3, dst3, ew3, zeros)
    return _final(p3, h2, W3_root.T, b3.reshape(1, -1))


# speedup vs baseline: 2.6791x; 2.6791x over previous
import functools

import jax
import jax.numpy as jnp
from jax import lax
from jax.experimental import pallas as pl
from jax.experimental.pallas import tpu as pltpu
from jax.experimental.pallas import tpu_sc as plsc

NC, NS, LANES = 2, 16, 16
NW = NC * NS
EB = 128
NB = 80
E_PAD = NW * NB * EB
N_NODES = 10000
N_PAD = 10240
RPT = N_PAD // NS
FC = 128


def _sc_agg(h, src3, dst3, ew3, zeros):
    mesh = plsc.VectorSubcoreMesh(
        core_axis_name="c", subcore_axis_name="s", num_cores=NC, num_subcores=NS)

    @functools.partial(
        pl.kernel,
        out_type=jax.ShapeDtypeStruct((NC, N_PAD, FC), jnp.float32),
        mesh=mesh,
        scratch_types=[
            pltpu.VMEM((NB, EB), jnp.int32),
            pltpu.VMEM((NB, EB), jnp.int32),
            pltpu.VMEM((NB, EB), jnp.float32),
            pltpu.VMEM((EB, FC), jnp.float32),
            pltpu.VMEM_SHARED((N_PAD, FC), jnp.float32),
            pltpu.SemaphoreType.DMA,
        ],
    )
    def k(h_hbm, src_hbm, dst_hbm, ew_hbm, z_hbm, out_hbm,
          src_v, dst_v, ew_v, rows_v, acc_sh, sem):
        cid = lax.axis_index("c")
        sid = lax.axis_index("s")
        wid = cid * NS + sid
        pltpu.sync_copy(src_hbm.at[wid], src_v)
        pltpu.sync_copy(dst_hbm.at[wid], dst_v)
        pltpu.sync_copy(ew_hbm.at[wid], ew_v)
        r0 = sid * RPT
        pltpu.sync_copy(z_hbm.at[pl.ds(r0, RPT)], acc_sh.at[pl.ds(r0, RPT)])
        plsc.subcore_barrier()

        def batch_body(b, carry):
            pltpu.async_copy(h_hbm.at[src_v.at[b]], rows_v, sem).wait()

            def e16_body(e16, c2):
                wv = ew_v[b, pl.ds(e16 * LANES, LANES)]
                for i in range(LANES):
                    e = e16 * LANES + i
                    w16 = jnp.broadcast_to(wv[i], (LANES,))
                    for j in range(FC // LANES):
                        sl = pl.ds(j * LANES, LANES)
                        rows_v[e, sl] = rows_v[e, sl] * w16
                return c2

            lax.fori_loop(0, EB // LANES, e16_body, 0)
            pltpu.sync_copy(rows_v, acc_sh.at[dst_v.at[b]], add=True)
            return carry

        lax.fori_loop(0, NB, batch_body, 0)
        plsc.subcore_barrier()
        pltpu.sync_copy(acc_sh.at[pl.ds(r0, RPT)],
                        out_hbm.at[cid, pl.ds(r0, RPT)])

    return k(h, src3, dst3, ew3, zeros)


def _dense2(parts, h, wa_t, wb_t, b2d, relu, bn=400):
    n, fin = h.shape
    fout = wa_t.shape[1]

    def body(p_ref, h_ref, wa_ref, wb_ref, b_ref, o_ref):
        agg = p_ref[0] + p_ref[1]
        z = jnp.dot(agg, wa_ref[...], preferred_element_type=jnp.float32,
                    precision=lax.Precision.HIGHEST)
        z = z + jnp.dot(h_ref[...], wb_ref[...], preferred_element_type=jnp.float32,
                        precision=lax.Precision.HIGHEST)
        z = z + b_ref[...]
        o_ref[...] = jnp.maximum(z, 0.0) if relu else z

    return pl.pallas_call(
        body,
        grid=(n // bn,),
        in_specs=[
            pl.BlockSpec((2, bn, fin), lambda i: (0, i, 0)),
            pl.BlockSpec((bn, fin), lambda i: (i, 0)),
            pl.BlockSpec((fin, fout), lambda i: (0, 0)),
            pl.BlockSpec((fin, fout), lambda i: (0, 0)),
            pl.BlockSpec((1, fout), lambda i: (0, 0)),
        ],
        out_specs=pl.BlockSpec((bn, fout), lambda i: (i, 0)),
        out_shape=jax.ShapeDtypeStruct((n, fout), jnp.float32),
    )(parts, h, wa_t, wb_t, b2d)


def _matmul(h, w_t, bn=400):
    n, fin = h.shape
    fout = w_t.shape[1]

    def body(h_ref, w_ref, o_ref):
        o_ref[...] = jnp.dot(h_ref[...], w_ref[...],
                             preferred_element_type=jnp.float32,
                             precision=lax.Precision.HIGHEST)

    return pl.pallas_call(
        body,
        grid=(n // bn,),
        in_specs=[
            pl.BlockSpec((bn, fin), lambda i: (i, 0)),
            pl.BlockSpec((fin, fout), lambda i: (0, 0)),
        ],
        out_specs=pl.BlockSpec((bn, fout), lambda i: (i, 0)),
        out_shape=jax.ShapeDtypeStruct((n, fout), jnp.float32),
    )(h, w_t)


def _final(parts, h, w_t, b2d, bn=400):
    n, fin = h.shape
    fout = w_t.shape[1]

    def body(p_ref, h_ref, w_ref, b_ref, o_ref):
        z = jnp.dot(h_ref[...], w_ref[...], preferred_element_type=jnp.float32,
                    precision=lax.Precision.HIGHEST)
        o_ref[...] = p_ref[0] + p_ref[1] + z + b_ref[...]

    return pl.pallas_call(
        body,
        grid=(n // bn,),
        in_specs=[
            pl.BlockSpec((2, bn, fout), lambda i: (0, i, 0)),
            pl.BlockSpec((bn, fin), lambda i: (i, 0)),
            pl.BlockSpec((fin, fout), lambda i: (0, 0)),
            pl.BlockSpec((1, fout), lambda i: (0, 0)),
        ],
        out_specs=pl.BlockSpec((bn, fout), lambda i: (i, 0)),
        out_shape=jax.ShapeDtypeStruct((n, fout), jnp.float32),
    )(parts, h, w_t, b2d)


def kernel(x, edge_index, edge_weight, W1_rel, b1, W1_root,
           W2_rel, b2, W2_root, W3_rel, b3, W3_root):
    src = edge_index[0]
    dst = edge_index[1]
    pad = E_PAD - src.shape[0]
    src3 = jnp.concatenate([src, jnp.zeros((pad,), src.dtype)]).reshape(NW, NB, EB)
    dst3 = jnp.concatenate([dst, jnp.zeros((pad,), dst.dtype)]).reshape(NW, NB, EB)
    ew3 = jnp.concatenate([edge_weight, jnp.zeros((pad,), edge_weight.dtype)]
                          ).reshape(NW, NB, EB)
    zeros = jnp.zeros((N_PAD, FC), jnp.float32)

    p1 = _sc_agg(x, src3, dst3, ew3, zeros)
    h1 = _dense2(p1, x, W1_rel.T, W1_root.T, b1.reshape(1, -1), relu=True)

    p2a = _sc_agg(h1[:, :FC], src3, dst3, ew3, zeros)
    p2b = _sc_agg(h1[:, FC:], src3, dst3, ew3, zeros)
    parts2 = jnp.concatenate([p2a, p2b], axis=2)
    h2 = _dense2(parts2, h1, W2_rel.T, W2_root.T, b2.reshape(1, -1), relu=True)

    y3 = _matmul(h2, W3_rel.T)
    p3 = _sc_agg(y3, src3, dst3, ew3, zeros)
    return _final(p3, h2, W3_root.T, b3.reshape(1, -1))


# speedup vs baseline: 3.2194x; 1.2017x over previous
import functools

import jax
import jax.numpy as jnp
from jax import lax
from jax.experimental import pallas as pl
from jax.experimental.pallas import tpu as pltpu
from jax.experimental.pallas import tpu_sc as plsc

NC, NS, LANES = 2, 16, 16
NW = NC * NS
EB = 128
NB = 80
E_PAD = NW * NB * EB
N_NODES = 10000
N_PAD = 10240
RPT = N_PAD // NS
FC = 128


def _sc_agg(h, e3, ew3, zeros):
    mesh = plsc.VectorSubcoreMesh(
        core_axis_name="c", subcore_axis_name="s", num_cores=NC, num_subcores=NS)

    @functools.partial(
        pl.kernel,
        out_type=jax.ShapeDtypeStruct((NC, N_PAD, FC), jnp.float32),
        mesh=mesh,
        scratch_types=[
            pltpu.VMEM((2, 2, EB), jnp.int32),
            pltpu.VMEM((2, EB), jnp.float32),
            pltpu.VMEM((2, EB, FC), jnp.float32),
            pltpu.VMEM_SHARED((N_PAD, FC), jnp.float32),
            pltpu.SemaphoreType.DMA,
        ],
    )
    def k(h_hbm, e3_hbm, ew_hbm, z_hbm, out_hbm, e3_v, ew_v, rows_v, acc_sh, gsem):
        cid = lax.axis_index("c")
        sid = lax.axis_index("s")
        wid = cid * NS + sid
        r0 = sid * RPT
        pltpu.sync_copy(z_hbm.at[pl.ds(r0, RPT)], acc_sh.at[pl.ds(r0, RPT)])

        def fetch_idx(b, buf):
            pltpu.sync_copy(e3_hbm.at[wid, b], e3_v.at[buf])
            pltpu.sync_copy(ew_hbm.at[wid, b], ew_v.at[buf])

        def gather(b, buf):
            return pltpu.make_async_copy(
                h_hbm.at[e3_v.at[buf, 0]], rows_v.at[buf], gsem)

        def scale(buf):
            def e16_body(e16, c2):
                wv = ew_v[buf, pl.ds(e16 * LANES, LANES)]
                for i in range(LANES):
                    e = e16 * LANES + i
                    w16 = jnp.broadcast_to(wv[i], (LANES,))
                    for j in range(FC // LANES):
                        sl = pl.ds(j * LANES, LANES)
                        rows_v[buf, e, sl] = rows_v[buf, e, sl] * w16
                return c2
            lax.fori_loop(0, EB // LANES, e16_body, 0)

        def scatter_add(buf):
            pltpu.sync_copy(rows_v.at[buf], acc_sh.at[e3_v.at[buf, 1]],
                            add=True)

        plsc.subcore_barrier()
        fetch_idx(0, 0)
        gather(0, 0).start()
        fetch_idx(1, 1)
        gather(1, 1).start()
        half = NB // 2

        def group_body(g, carry):
            for buf in (0, 1):
                b = 2 * g + buf
                gather(b, buf).wait()
                scale(buf)
                scatter_add(buf)

                @pl.when(g < half - 1)
                def _():
                    fetch_idx(b + 2, buf)
                    gather(b + 2, buf).start()
            return carry

        lax.fori_loop(0, half, group_body, 0)
        plsc.subcore_barrier()
        pltpu.sync_copy(acc_sh.at[pl.ds(r0, RPT)],
                        out_hbm.at[cid, pl.ds(r0, RPT)])

    return k(h, e3, ew3, zeros)


def _dense2(parts, h, wa_t, wb_t, b2d, relu, bn=400):
    n, fin = h.shape
    fout = wa_t.shape[1]

    def body(p_ref, h_ref, wa_ref, wb_ref, b_ref, o_ref):
        agg = p_ref[0] + p_ref[1]
        z = jnp.dot(agg, wa_ref[...], preferred_element_type=jnp.float32,
                    precision=lax.Precision.HIGHEST)
        z = z + jnp.dot(h_ref[...], wb_ref[...], preferred_element_type=jnp.float32,
                        precision=lax.Precision.HIGHEST)
        z = z + b_ref[...]
        o_ref[...] = jnp.maximum(z, 0.0) if relu else z

    return pl.pallas_call(
        body,
        grid=(n // bn,),
        in_specs=[
            pl.BlockSpec((2, bn, fin), lambda i: (0, i, 0)),
            pl.BlockSpec((bn, fin), lambda i: (i, 0)),
            pl.BlockSpec((fin, fout), lambda i: (0, 0)),
            pl.BlockSpec((fin, fout), lambda i: (0, 0)),
            pl.BlockSpec((1, fout), lambda i: (0, 0)),
        ],
        out_specs=pl.BlockSpec((bn, fout), lambda i: (i, 0)),
        out_shape=jax.ShapeDtypeStruct((n, fout), jnp.float32),
    )(parts, h, wa_t, wb_t, b2d)


def _matmul(h, w_t, bn=400):
    n, fin = h.shape
    fout = w_t.shape[1]

    def body(h_ref, w_ref, o_ref):
        o_ref[...] = jnp.dot(h_ref[...], w_ref[...],
                             preferred_element_type=jnp.float32,
                             precision=lax.Precision.HIGHEST)

    return pl.pallas_call(
        body,
        grid=(n // bn,),
        in_specs=[
            pl.BlockSpec((bn, fin), lambda i: (i, 0)),
            pl.BlockSpec((fin, fout), lambda i: (0, 0)),
        ],
        out_specs=pl.BlockSpec((bn, fout), lambda i: (i, 0)),
        out_shape=jax.ShapeDtypeStruct((n, fout), jnp.float32),
    )(h, w_t)


def _final(parts, h, w_t, b2d, bn=400):
    n, fin = h.shape
    fout = w_t.shape[1]

    def body(p_ref, h_ref, w_ref, b_ref, o_ref):
        z = jnp.dot(h_ref[...], w_ref[...], preferred_element_type=jnp.float32,
                    precision=lax.Precision.HIGHEST)
        o_ref[...] = p_ref[0] + p_ref[1] + z + b_ref[...]

    return pl.pallas_call(
        body,
        grid=(n // bn,),
        in_specs=[
            pl.BlockSpec((2, bn, fout), lambda i: (0, i, 0)),
            pl.BlockSpec((bn, fin), lambda i: (i, 0)),
            pl.BlockSpec((fin, fout), lambda i: (0, 0)),
            pl.BlockSpec((1, fout), lambda i: (0, 0)),
        ],
        out_specs=pl.BlockSpec((bn, fout), lambda i: (i, 0)),
        out_shape=jax.ShapeDtypeStruct((n, fout), jnp.float32),
    )(parts, h, w_t, b2d)


def kernel(x, edge_index, edge_weight, W1_rel, b1, W1_root,
           W2_rel, b2, W2_root, W3_rel, b3, W3_root):
    src = edge_index[0]
    dst = edge_index[1]
    pad = E_PAD - src.shape[0]
    src_r = jnp.concatenate([src, jnp.zeros((pad,), jnp.int32)]).reshape(NW, NB, EB)
    dst_r = jnp.concatenate([dst, jnp.zeros((pad,), jnp.int32)]).reshape(NW, NB, EB)
    ew3 = jnp.concatenate([edge_weight, jnp.zeros((pad,), jnp.float32)]).reshape(NW, NB, EB)
    e3 = jnp.stack([src_r, dst_r], axis=2)
    zeros = jnp.zeros((N_PAD, FC), jnp.float32)

    p1 = _sc_agg(x, e3, ew3, zeros)
    h1 = _dense2(p1, x, W1_rel.T, W1_root.T, b1.reshape(1, -1), relu=True)

    p2a = _sc_agg(h1[:, :FC], e3, ew3, zeros)
    p2b = _sc_agg(h1[:, FC:], e3, ew3, zeros)
    parts2 = jnp.concatenate([p2a, p2b], axis=2)
    h2 = _dense2(parts2, h1, W2_rel.T, W2_root.T, b2.reshape(1, -1), relu=True)

    y3 = _matmul(h2, W3_rel.T)
    p3 = _sc_agg(y3, e3, ew3, zeros)
    return _final(p3, h2, W3_root.T, b3.reshape(1, -1))
